# baseline (speedup 1.0000x reference)
import jax
import jax.numpy as jnp
from jax import lax
from jax.experimental import pallas as pl
from jax.experimental.pallas import tpu as pltpu

_PAD = 1
_BB = 8
_H = 14
_W = 14


_NSPLIT = 4


def _in_copy(x_hbm, sin, in_sems, step, slot):
    bn = _BB // _NSPLIT
    return [pltpu.make_async_copy(
        x_hbm.at[pl.ds(step * _BB + k * bn, bn)],
        sin.at[slot, pl.ds(k * bn, bn)],
        in_sems.at[slot],
    ) for k in range(_NSPLIT)]


def _out_copy(o_hbm, sout, out_sems, step, slot):
    bn = _BB // _NSPLIT
    return [pltpu.make_async_copy(
        sout.at[slot, pl.ds(k * bn, bn)],
        o_hbm.at[pl.ds(step * _BB + k * bn, bn)],
        out_sems.at[slot],
    ) for k in range(_NSPLIT)]


def _pad_kernel(x_hbm, o_hbm, sin, sout, in_sems, out_sems):
    i = pl.program_id(0)
    nb = pl.num_programs(0)
    slot = lax.rem(i, 2)
    c = sout.shape[2]
    hp = _H + 2 * _PAD
    wp = _W + 2 * _PAD

    @pl.when(i == 0)
    def _prologue():
        zrow = jnp.zeros((2, _BB, c, wp), sout.dtype)
        zcol = jnp.zeros((2, _BB, c, hp), sout.dtype)
        sout[:, :, :, 0, :] = zrow
        sout[:, :, :, hp - 1, :] = zrow
        sout[:, :, :, :, 0] = zcol
        sout[:, :, :, :, wp - 1] = zcol
        [c.start() for c in _in_copy(x_hbm, sin, in_sems, 0, 0)]

    [c.wait() for c in _in_copy(x_hbm, sin, in_sems, i, slot)]

    @pl.when(i >= 2)
    def _drain_reader():
        [c.wait() for c in _out_copy(o_hbm, sout, out_sems, i - 2, slot)]

    sout[slot, :, :, _PAD:_PAD + _H, _PAD:_PAD + _W] = sin[slot]

    [c.start() for c in _out_copy(o_hbm, sout, out_sems, i, slot)]

    @pl.when(i + 1 < nb)
    def _prefetch():
        [c.start() for c in _in_copy(x_hbm, sin, in_sems, i + 1, 1 - slot)]

    @pl.when(i == nb - 1)
    def _epilogue():
        @pl.when(i >= 1)
        def _drain_prev():
            [c.wait() for c in _out_copy(o_hbm, sout, out_sems, i - 1, 1 - slot)]
        [c.wait() for c in _out_copy(o_hbm, sout, out_sems, i, slot)]


def kernel(x):
    b, c, h, w = x.shape
    hp, wp = h + 2 * _PAD, w + 2 * _PAD
    return pl.pallas_call(
        _pad_kernel,
        grid=(b // _BB,),
        in_specs=[pl.BlockSpec(memory_space=pl.ANY)],
        out_specs=pl.BlockSpec(memory_space=pl.ANY),
        out_shape=jax.ShapeDtypeStruct((b, c, hp, wp), x.dtype),
        scratch_shapes=[
            pltpu.VMEM((2, _BB, c, h, w), x.dtype),
            pltpu.VMEM((2, _BB, c, hp, wp), x.dtype),
            pltpu.SemaphoreType.DMA((2,)),
            pltpu.SemaphoreType.DMA((2,)),
        ],
        compiler_params=pltpu.CompilerParams(
            dimension_semantics=("arbitrary",)),
    )(x)


# speedup vs baseline: 1.2023x; 1.2023x over previous
import jax
import jax.numpy as jnp
from jax.experimental import pallas as pl
from jax.experimental.pallas import tpu as pltpu

_PAD = 1
_BB = 16


def _pad_kernel(x_ref, o_ref):
    o_ref[...] = jnp.zeros_like(o_ref)
    o_ref[:, :, _PAD:_PAD + 14, _PAD:_PAD + 14] = x_ref[...]


def kernel(x):
    b, c, h, w = x.shape
    return pl.pallas_call(
        _pad_kernel,
        grid=(b // _BB,),
        in_specs=[pl.BlockSpec((_BB, c, h, w), lambda i: (i, 0, 0, 0))],
        out_specs=pl.BlockSpec((_BB, c, h + 2 * _PAD, w + 2 * _PAD),
                               lambda i: (i, 0, 0, 0)),
        out_shape=jax.ShapeDtypeStruct((b, c, h + 2 * _PAD, w + 2 * _PAD),
                                       x.dtype),
        compiler_params=pltpu.CompilerParams(
            dimension_semantics=("arbitrary",),
            vmem_limit_bytes=100 * 1024 * 1024),
    )(x)
